# baseline (speedup 1.0000x reference)
import functools

import jax
import jax.numpy as jnp
from jax import lax
from jax.experimental import pallas as pl
from jax.experimental.pallas import tpu as pltpu
from jax.experimental.pallas import tpu_sc as plsc

_K = 4
_ROWS_PER_BLOCK = 1024
_GATHER_WINDOW = 128


def _dist_topk_body(x_ref, e_ref, idx_ref, diff_ref, cbt_ref, qst_ref, *,
                    denom):
    pid = pl.program_id(0)
    nprog = pl.num_programs(0)
    x = x_ref[...]
    e = e_ref[...]
    mm = lax.dot_general(x, e, (((1,), (0,)), ((), ())),
                         preferred_element_type=jnp.float32)
    x2 = jnp.sum(x * x, axis=1, keepdims=True)
    e2 = jnp.sum(e * e, axis=0, keepdims=True)
    dist = (x2 - 2.0 * mm) + e2
    col = lax.broadcasted_iota(jnp.int32, dist.shape, 1)
    big = jnp.int32(2 ** 30)
    idx_cols = []
    bsum = jnp.float32(0.0)
    onehot = None
    for k in range(_K):
        m = jnp.min(dist, axis=1, keepdims=True)
        idxk = jnp.min(jnp.where(dist == m, col, big), axis=1)
        idx_cols.append(idxk)
        if k == 0:
            bsum = jnp.sum(m)
            onehot = (col == idxk[:, None]).astype(jnp.float32)
        if k < _K - 1:
            dist = jnp.where(col == idxk[:, None], jnp.inf, dist)
    idx_ref[...] = jnp.stack(idx_cols, axis=1)
    q = lax.dot_general(onehot, e, (((1,), (1,)), ((), ())),
                        precision=lax.Precision.HIGHEST,
                        preferred_element_type=jnp.float32)
    qst_ref[...] = x + (q - x)
    acc = jnp.where(pid == 0, 0.0, diff_ref[0, 0]) + bsum
    diff_ref[0, 0] = jnp.where(pid == nprog - 1, acc / denom, acc)

    @pl.when(pid == 0)
    def _():
        cbt_ref[...] = e.T


def _dist_topk(flat, embed):
    n_rows, dim = flat.shape
    n_embed = embed.shape[1]
    r = _ROWS_PER_BLOCK
    grid = n_rows // r
    return pl.pallas_call(
        functools.partial(_dist_topk_body, denom=float(n_rows * dim)),
        grid=(grid,),
        in_specs=[
            pl.BlockSpec((r, dim), lambda i: (i, 0)),
            pl.BlockSpec((dim, n_embed), lambda i: (0, 0)),
        ],
        out_specs=[
            pl.BlockSpec((r, _K), lambda i: (i, 0)),
            pl.BlockSpec((1, 1), lambda i: (0, 0), memory_space=pltpu.SMEM),
            pl.BlockSpec((n_embed, dim), lambda i: (0, 0)),
            pl.BlockSpec((r, dim), lambda i: (i, 0)),
        ],
        out_shape=[
            jax.ShapeDtypeStruct((n_rows, _K), jnp.int32),
            jax.ShapeDtypeStruct((1, 1), jnp.float32),
            jax.ShapeDtypeStruct((n_embed, dim), jnp.float32),
            jax.ShapeDtypeStruct((n_rows, dim), jnp.float32),
        ],
    )(flat, embed)


def _sc_gather(cbt, idx_flat):
    n = idx_flat.shape[1]
    dim = cbt.shape[1]
    gw = _GATHER_WINDOW
    mesh = plsc.VectorSubcoreMesh(core_axis_name="core",
                                  subcore_axis_name="subcore")

    @functools.partial(
        pl.kernel,
        out_type=jax.ShapeDtypeStruct((n, dim), jnp.float32),
        mesh=mesh,
        compiler_params=pltpu.CompilerParams(use_tc_tiling_on_sc=False),
    )
    def gk(cbt_hbm, i_hbm, o_hbm):
        def body(i_vmem, o_vmem):
            pltpu.sync_copy(cbt_hbm.at[i_vmem.at[0]], o_vmem)

        pltpu.emit_pipeline(
            body,
            grid=(n // gw,),
            in_specs=[pl.BlockSpec((1, gw), lambda i: (0, i))],
            out_specs=[pl.BlockSpec((gw, dim), lambda i: (i, 0))],
            core_axis_name=("core", "subcore"),
            dimension_semantics=(pltpu.PARALLEL,),
        )(i_hbm, o_hbm)

    return gk(cbt, idx_flat)


def kernel(input, embed):
    b, h, w, dim = input.shape
    flat = input.reshape(-1, dim)
    n_rows = flat.shape[0]
    idx, diff_acc, cbt, qst = _dist_topk(flat, embed)
    gathered = _sc_gather(cbt, idx.reshape(1, n_rows * _K))
    quantize_topk = gathered.reshape(b, h, w, _K * dim)
    diff = diff_acc[0, 0]
    quantize_st = qst.reshape(b, h, w, dim)
    return (quantize_topk, diff, quantize_st)


# speedup vs baseline: 1.3059x; 1.3059x over previous
import functools

import jax
import jax.numpy as jnp
from jax import lax
from jax.experimental import pallas as pl
from jax.experimental.pallas import tpu as pltpu
from jax.experimental.pallas import tpu_sc as plsc

_K = 4
_ROWS_PER_BLOCK = 1024
_GATHER_WINDOW = 512
_SUB_ROWS = 128


def _dist_topk_body(x_ref, e_ref, idx_ref, diff_ref, cbt_ref, qst_ref, *,
                    denom):
    pid = pl.program_id(0)
    nprog = pl.num_programs(0)
    x = x_ref[...]
    e = e_ref[...]
    mm = lax.dot_general(x, e, (((1,), (0,)), ((), ())),
                         preferred_element_type=jnp.float32)
    x2 = jnp.sum(x * x, axis=1, keepdims=True)
    e2 = jnp.sum(e * e, axis=0, keepdims=True)
    dist = (x2 - 2.0 * mm) + e2
    colf = lax.broadcasted_iota(jnp.int32, dist.shape, 1).astype(jnp.float32)
    inf = jnp.float32(jnp.inf)
    idx_cols = []
    bsum = jnp.float32(0.0)
    onehot = None
    for k in range(_K):
        m = jnp.min(dist, axis=1, keepdims=True)
        idxf = jnp.min(jnp.where(dist == m, colf, inf), axis=1)
        idx_cols.append(idxf.astype(jnp.int32))
        if k == 0:
            bsum = jnp.sum(m)
            onehot = (colf == idxf[:, None]).astype(jnp.bfloat16)
        if k < _K - 1:
            dist = jnp.where(colf == idxf[:, None], inf, dist)
    idx_ref[...] = jnp.stack(idx_cols, axis=1)
    e_hi = e.astype(jnp.bfloat16)
    e_lo = (e - e_hi.astype(jnp.float32)).astype(jnp.bfloat16)
    q = (lax.dot_general(onehot, e_hi, (((1,), (1,)), ((), ())),
                         preferred_element_type=jnp.float32)
         + lax.dot_general(onehot, e_lo, (((1,), (1,)), ((), ())),
                           preferred_element_type=jnp.float32))
    qst_ref[...] = x + (q - x)
    acc = jnp.where(pid == 0, 0.0, diff_ref[0, 0]) + bsum
    diff_ref[0, 0] = jnp.where(pid == nprog - 1, acc / denom, acc)

    @pl.when(pid == 0)
    def _():
        cbt_ref[...] = e.T


def _dist_topk(flat, embed):
    n_rows, dim = flat.shape
    n_embed = embed.shape[1]
    r = _ROWS_PER_BLOCK
    grid = n_rows // r
    return pl.pallas_call(
        functools.partial(_dist_topk_body, denom=float(n_rows * dim)),
        grid=(grid,),
        in_specs=[
            pl.BlockSpec((r, dim), lambda i: (i, 0)),
            pl.BlockSpec((dim, n_embed), lambda i: (0, 0)),
        ],
        out_specs=[
            pl.BlockSpec((r, _K), lambda i: (i, 0)),
            pl.BlockSpec((1, 1), lambda i: (0, 0), memory_space=pltpu.SMEM),
            pl.BlockSpec((n_embed, dim), lambda i: (0, 0)),
            pl.BlockSpec((r, dim), lambda i: (i, 0)),
        ],
        out_shape=[
            jax.ShapeDtypeStruct((n_rows, _K), jnp.int32),
            jax.ShapeDtypeStruct((1, 1), jnp.float32),
            jax.ShapeDtypeStruct((n_embed, dim), jnp.float32),
            jax.ShapeDtypeStruct((n_rows, dim), jnp.float32),
        ],
    )(flat, embed)


def _sc_gather(cbt, idx_flat):
    n = idx_flat.shape[1]
    dim = cbt.shape[1]
    gw = _GATHER_WINDOW
    sub = gw // _SUB_ROWS
    mesh = plsc.VectorSubcoreMesh(core_axis_name="core",
                                  subcore_axis_name="subcore")

    @functools.partial(
        pl.kernel,
        out_type=jax.ShapeDtypeStruct((n, dim), jnp.float32),
        mesh=mesh,
        scratch_types=[pltpu.SemaphoreType.DMA],
        compiler_params=pltpu.CompilerParams(use_tc_tiling_on_sc=False),
    )
    def gk(cbt_hbm, i_hbm, o_hbm, sem):
        def body(i_vmem, o_vmem):
            copies = [
                pltpu.async_copy(
                    cbt_hbm.at[i_vmem.at[0, pl.ds(j * _SUB_ROWS, _SUB_ROWS)]],
                    o_vmem.at[pl.ds(j * _SUB_ROWS, _SUB_ROWS)],
                    sem,
                )
                for j in range(sub)
            ]
            for c in copies:
                c.wait()

        pltpu.emit_pipeline(
            body,
            grid=(n // gw,),
            in_specs=[pl.BlockSpec((1, gw), lambda i: (0, i))],
            out_specs=[pl.BlockSpec((gw, dim), lambda i: (i, 0))],
            core_axis_name=("core", "subcore"),
            dimension_semantics=(pltpu.PARALLEL,),
        )(i_hbm, o_hbm)

    return gk(cbt, idx_flat)


def kernel(input, embed):
    b, h, w, dim = input.shape
    flat = input.reshape(-1, dim)
    n_rows = flat.shape[0]
    idx, diff_acc, cbt, qst = _dist_topk(flat, embed)
    gathered = _sc_gather(cbt, idx.reshape(1, n_rows * _K))
    quantize_topk = gathered.reshape(b, h, w, _K * dim)
    diff = diff_acc[0, 0]
    quantize_st = qst.reshape(b, h, w, dim)
    return (quantize_topk, diff, quantize_st)
